# baseline (speedup 1.0000x reference)
import jax
import jax.numpy as jnp
from jax.experimental import pallas as pl

N0, N1, N2 = 50000, 12500, 3125
LATENT, HID = 128, 64


def _leaky(x):
    return jnp.where(x > 0, x, 0.01 * x)


def _mpl(x, ei, ea, p):
    src, dst = ei[0], ei[1]
    m = jnp.concatenate([x[src], x[dst], ea], axis=-1)
    e = _leaky(m @ p['We1']) @ p['We2']
    agg = jax.ops.segment_sum(e, dst, num_segments=x.shape[0])
    h = jnp.concatenate([x, agg], axis=-1)
    xo = _leaky(h @ p['Wn1']) @ p['Wn2']
    return xo, e


def _unpool(h, n, idx):
    return jnp.zeros((n, h.shape[-1]), h.dtype).at[idx].set(h)


def _res_up(x, ea, ei_coarse, ei_fine, m_id, e_idx, up_nodes, p):
    xs = _unpool(x, up_nodes, m_id)
    eas = jnp.zeros((ei_fine.shape[1], ea.shape[-1]), ea.dtype).at[e_idx].set(ea)
    xs, _ = _mpl(xs, ei_fine, eas, p['skip'])
    xm, eam = _mpl(x, ei_coarse, ea, p['mpl1'])
    xm = _unpool(xm, up_nodes, m_id)
    eam = jnp.zeros((ei_fine.shape[1], eam.shape[-1]), eam.dtype).at[e_idx].set(eam)
    xm, eam = _mpl(xm, ei_fine, eam, p['mpl2'])
    return _leaky(xm + xs), eam


def _ln(v, g, b, eps=1e-5):
    mu = jnp.mean(v, axis=-1, keepdims=True)
    var = jnp.var(v, axis=-1, keepdims=True)
    return (v - mu) / jnp.sqrt(var + eps) * g + b


def _pl_leaky(x):
    def body(x_ref, o_ref):
        v = x_ref[...]
        o_ref[...] = jnp.where(v > 0, v, 0.01 * v)
    n, d = x.shape
    blk = 8000
    return pl.pallas_call(
        body,
        grid=(pl.cdiv(n, blk),),
        in_specs=[pl.BlockSpec((blk, d), lambda i: (i, 0))],
        out_specs=pl.BlockSpec((blk, d), lambda i: (i, 0)),
        out_shape=jax.ShapeDtypeStruct(x.shape, x.dtype),
    )(x)


def kernel(z, edge_index_l0, edge_index_l1, edge_index_l2, m_id_0, m_id_1, e_idx_0, e_idx_1, params):
    px = params['up_x']
    x = _leaky(z @ px['W1'] + px['b1']) @ px['W2'] + px['b2']
    pe = params['up_e']
    e = _leaky(z @ pe['W1'] + pe['b1']) @ pe['W2'] + pe['b2']
    x = jnp.transpose(x, (0, 2, 1)).reshape(-1, LATENT)
    e = jnp.transpose(e, (0, 2, 1)).reshape(-1, LATENT)
    x, e = _mpl(x, edge_index_l2, e, params['bottom'])
    x, e = _res_up(x, e, edge_index_l2, edge_index_l1, m_id_1, e_idx_1, N1, params['l0'])
    x, e = _res_up(x, e, edge_index_l1, edge_index_l0, m_id_0, e_idx_0, N0, params['l1'])
    x, e = _mpl(x, edge_index_l0, e, params['final'])
    pn = params['out_n']
    xo = _pl_leaky(x @ pn['W1'] + pn['b1']) @ pn['W2'] + pn['b2']
    xo = _ln(xo, pn['g'], pn['b'])
    po = params['out_e']
    eo = _pl_leaky(e @ po['W1'] + po['b1']) @ po['W2'] + po['b2']
    eo = _ln(eo, po['g'], po['b'])
    return xo, eo
